# Initial kernel scaffold; baseline (speedup 1.0000x reference)
import jax
import jax.numpy as jnp
from jax.experimental import pallas as pl


def kernel(x, edge_index, batch, bn_w, bn_b, bn_m, bn_v, W, b, fc_bn_w, fc_bn_b, fc_bn_m, fc_bn_v, fc_W, fc_b, cls_W, cls_b):
    raise NotImplementedError("write your pallas kernel here")


# speedup vs baseline: 6.7389x; 6.7389x over previous
import functools

import jax
import jax.numpy as jnp
from jax import lax
from jax.experimental import pallas as pl
from jax.experimental.pallas import tpu as pltpu
from jax.experimental.pallas import tpu_sc as plsc

N = 10000
D = 128
H = 128
OUT = 10
G = 64
L = 3
E = 320000

NC = 2
NS = 16
NW = NC * NS

EPW = 10240
E_PAD = EPW * NW
NG = EPW // 128
N_ACC = 10240
SLC = N_ACC // NS

_mesh = plsc.VectorSubcoreMesh(core_axis_name="c", subcore_axis_name="s")

F32 = jnp.float32
HIGHEST = lax.Precision.HIGHEST


@functools.partial(
    pl.kernel,
    mesh=_mesh,
    out_type=jax.ShapeDtypeStruct((NC, N_ACC), F32),
    scratch_types=[
        pltpu.VMEM((NG, 128), jnp.int32),
        pltpu.VMEM((128,), F32),
        pltpu.VMEM((SLC,), F32),
        pltpu.VMEM_SHARED((N_ACC,), F32),
    ],
)
def _deg_pass(dsti_hbm, out_hbm, dst_v, ones_v, zb_v, acc_sh):
    c = lax.axis_index("c")
    s = lax.axis_index("s")
    wid = c * NS + s

    def _fill_z(i, _):
        zb_v[pl.ds(i * 16, 16)] = jnp.zeros((16,), F32)
        return 0

    lax.fori_loop(0, SLC // 16, _fill_z, 0)
    for j in range(8):
        ones_v[pl.ds(j * 16, 16)] = jnp.ones((16,), F32)
    pltpu.sync_copy(zb_v, acc_sh.at[pl.ds(s * SLC, SLC)])
    plsc.subcore_barrier()

    pltpu.sync_copy(dsti_hbm.at[wid], dst_v)

    def _scatter(g, _):
        pltpu.sync_copy(ones_v, acc_sh.at[dst_v.at[g]], add=True)
        return 0

    lax.fori_loop(0, NG, _scatter, 0)
    plsc.subcore_barrier()
    pltpu.sync_copy(acc_sh.at[pl.ds(s * SLC, SLC)],
                    out_hbm.at[c, pl.ds(s * SLC, SLC)])


@functools.partial(
    pl.kernel,
    mesh=_mesh,
    out_type=jax.ShapeDtypeStruct((NC, N_ACC, 128), F32),
    scratch_types=[
        pltpu.VMEM((NG, 128), jnp.int32),
        pltpu.VMEM((NG, 128), jnp.int32),
        pltpu.VMEM((128, 128), F32),
        pltpu.SemaphoreType.DMA,
        pltpu.VMEM_SHARED((N_ACC, 128), F32),
    ],
)
def _sc_scatter(table_hbm, srci_hbm, dsti_hbm, out_hbm,
                src_v, dst_v, rows_a, sem_a, acc_sh):
    c = lax.axis_index("c")
    s = lax.axis_index("s")
    wid = c * NS + s

    def _zrow(i, _):
        for j in range(8):
            rows_a[i, pl.ds(j * 16, 16)] = jnp.zeros((16,), F32)
        return 0

    lax.fori_loop(0, 128, _zrow, 0)
    for r in range(SLC // 128):
        pltpu.sync_copy(rows_a, acc_sh.at[pl.ds(s * SLC + r * 128, 128)])
    plsc.subcore_barrier()

    pltpu.sync_copy(srci_hbm.at[wid], src_v)
    pltpu.sync_copy(dsti_hbm.at[wid], dst_v)

    def _edge_group(g, _):
        pltpu.async_copy(table_hbm.at[src_v.at[g]], rows_a, sem_a).wait()
        pltpu.sync_copy(rows_a, acc_sh.at[dst_v.at[g]], add=True)
        return 0

    lax.fori_loop(0, NG, _edge_group, 0)
    plsc.subcore_barrier()
    pltpu.sync_copy(acc_sh.at[pl.ds(s * SLC, SLC)],
                    out_hbm.at[c, pl.ds(s * SLC, SLC)])


def _dinv_body(degp_ref, out_ref):
    deg = degp_ref[0, :] + degp_ref[1, :] + 1.0
    out_ref[...] = lax.rsqrt(deg)


def _dinv_tc(deg_p):
    return pl.pallas_call(
        _dinv_body,
        out_shape=jax.ShapeDtypeStruct((N_ACC,), F32),
    )(deg_p)


BN_ROWS = 1000


def _bn_scale(bw, bb, bm, bv):
    s = bw * lax.rsqrt(bv + 1e-5)
    return s, bb - bm * s


def _prep_body(x_ref, dinv_ref, bw_ref, bb_ref, bm_ref, bv_ref, w_ref, out_ref):
    s, t = _bn_scale(bw_ref[...], bb_ref[...], bm_ref[...], bv_ref[...])
    h = x_ref[...] * s + t
    hw = jnp.dot(h, w_ref[...], preferred_element_type=F32, precision=HIGHEST)
    out_ref[...] = hw * dinv_ref[...]


def _prep_tc(x, dinv_col, bw, bb, bm, bv, w):
    grid = (N // BN_ROWS,)
    blk = lambda r, c_: pl.BlockSpec((r, c_), lambda i: (i, 0))
    full = lambda shape: pl.BlockSpec(shape, lambda i: (0,) * len(shape))
    return pl.pallas_call(
        _prep_body,
        grid=grid,
        in_specs=[blk(BN_ROWS, 128), blk(BN_ROWS, 1),
                  full((128,)), full((128,)), full((128,)), full((128,)),
                  full((128, 128))],
        out_specs=blk(BN_ROWS, 128),
        out_shape=jax.ShapeDtypeStruct((N, 128), F32),
    )(x, dinv_col, bw, bb, bm, bv, w)


def _combine_prep_body(with_residual,
                       p0_ref, p1_ref, hws_ref, prev_ref, dinv_ref, bias_ref,
                       bw_ref, bb_ref, bm_ref, bv_ref, w_ref,
                       out_ref, hwsn_ref):
    acc = (p0_ref[...] + p1_ref[...] + hws_ref[...]) * dinv_ref[...]
    o = jnp.maximum(acc + bias_ref[...], 0.0)
    if with_residual:
        o = o + prev_ref[...]
    out_ref[...] = o
    s, t = _bn_scale(bw_ref[...], bb_ref[...], bm_ref[...], bv_ref[...])
    h = o * s + t
    hw = jnp.dot(h, w_ref[...], preferred_element_type=F32, precision=HIGHEST)
    hwsn_ref[...] = hw * dinv_ref[...]


def _combine_prep_tc(with_residual, p0, p1, hws, prev, dinv_col, bias,
                     bw, bb, bm, bv, w):
    grid = (N // BN_ROWS,)
    blk = lambda r, c_: pl.BlockSpec((r, c_), lambda i: (i, 0))
    full = lambda shape: pl.BlockSpec(shape, lambda i: (0,) * len(shape))
    return pl.pallas_call(
        functools.partial(_combine_prep_body, with_residual),
        grid=grid,
        in_specs=[blk(BN_ROWS, 128), blk(BN_ROWS, 128), blk(BN_ROWS, 128),
                  blk(BN_ROWS, 128), blk(BN_ROWS, 1), full((128,)),
                  full((128,)), full((128,)), full((128,)), full((128,)),
                  full((128, 128))],
        out_specs=[blk(BN_ROWS, 128), blk(BN_ROWS, 128)],
        out_shape=[jax.ShapeDtypeStruct((N, 128), F32),
                   jax.ShapeDtypeStruct((N, 128), F32)],
    )(p0, p1, hws, prev, dinv_col, bias, bw, bb, bm, bv, w)


def _pool_head_body(p0_ref, p1_ref, hws_ref, prev_ref, dinv_ref, bias_ref,
                    batch_ref,
                    fbw_ref, fbb_ref, fbm_ref, fbv_ref,
                    fw_ref, fb_ref, cw_ref, cb_ref,
                    out_ref, sums_ref, cnts_ref):
    b = pl.program_id(0)

    @pl.when(b == 0)
    def _():
        sums_ref[...] = jnp.zeros((G, 128), F32)
        cnts_ref[...] = jnp.zeros((G, 128), F32)

    acc = (p0_ref[...] + p1_ref[...] + hws_ref[...]) * dinv_ref[...]
    o = jnp.maximum(acc + bias_ref[...], 0.0) + prev_ref[...]

    oh = (lax.broadcasted_iota(jnp.int32, (G, BN_ROWS), 0)
          == batch_ref[0]).astype(F32)
    sums_ref[...] += jnp.dot(oh, o, preferred_element_type=F32,
                             precision=HIGHEST)
    cnts_ref[...] += jnp.dot(oh, jnp.ones((BN_ROWS, 128), F32),
                             preferred_element_type=F32, precision=HIGHEST)

    @pl.when(b == N // BN_ROWS - 1)
    def _():
        pooled = sums_ref[...] / jnp.maximum(cnts_ref[...], 1.0)
        s, t = _bn_scale(fbw_ref[...], fbb_ref[...], fbm_ref[...], fbv_ref[...])
        h = pooled * s + t
        h = jnp.maximum(
            jnp.dot(h, fw_ref[...], preferred_element_type=F32,
                    precision=HIGHEST) + fb_ref[...], 0.0)
        lg = jnp.dot(h, cw_ref[...], preferred_element_type=F32,
                     precision=HIGHEST) + cb_ref[...]
        m = jnp.max(lg, axis=-1, keepdims=True)
        z = lg - m
        out_ref[...] = z - jnp.log(jnp.sum(jnp.exp(z), axis=-1, keepdims=True))


def _pool_head_tc(p0, p1, hws, prev, dinv_col, bias, batch,
                  fbw, fbb, fbm, fbv, fw, fb, cw_pad, cb_pad):
    grid = (N // BN_ROWS,)
    blk = lambda r, c_: pl.BlockSpec((r, c_), lambda i: (i, 0))
    full = lambda shape: pl.BlockSpec(shape, lambda i: (0,) * len(shape))
    return pl.pallas_call(
        _pool_head_body,
        grid=grid,
        in_specs=[blk(BN_ROWS, 128), blk(BN_ROWS, 128), blk(BN_ROWS, 128),
                  blk(BN_ROWS, 128), blk(BN_ROWS, 1), full((128,)),
                  pl.BlockSpec((1, 1, BN_ROWS), lambda i: (i, 0, 0)),
                  full((128,)), full((128,)), full((128,)), full((128,)),
                  full((128, 128)), full((128,)),
                  full((128, 128)), full((128,))],
        out_specs=pl.BlockSpec((G, 128), lambda i: (0, 0)),
        out_shape=jax.ShapeDtypeStruct((G, 128), F32),
        scratch_shapes=[pltpu.VMEM((G, 128), F32), pltpu.VMEM((G, 128), F32)],
    )(p0, p1, hws, prev, dinv_col, bias, batch,
      fbw, fbb, fbm, fbv, fw, fb, cw_pad, cb_pad)


def kernel(x, edge_index, batch, bn_w, bn_b, bn_m, bn_v, W, b,
           fc_bn_w, fc_bn_b, fc_bn_m, fc_bn_v, fc_W, fc_b, cls_W, cls_b):
    pad = E_PAD - E
    src_p = jnp.concatenate(
        [edge_index[0], jnp.zeros((pad,), jnp.int32)]).reshape(NW, NG, 128)
    dst_p = jnp.concatenate(
        [edge_index[1], jnp.full((pad,), N, jnp.int32)]).reshape(NW, NG, 128)

    deg_p = _deg_pass(dst_p)
    dinv_col = _dinv_tc(deg_p).reshape(N_ACC, 1)[:N]

    hws = _prep_tc(x, dinv_col, bn_w[0], bn_b[0], bn_m[0], bn_v[0], W[0])
    prev = x
    outs = None
    for i in range(L - 1):
        parts = _sc_scatter(hws, src_p, dst_p)
        outs = _combine_prep_tc(i > 0, parts[0, :N], parts[1, :N], hws, prev,
                                dinv_col, b[i],
                                bn_w[i + 1], bn_b[i + 1], bn_m[i + 1],
                                bn_v[i + 1], W[i + 1])
        prev, hws = outs

    parts = _sc_scatter(hws, src_p, dst_p)
    cw_pad = jnp.pad(cls_W, ((0, 0), (0, 128 - OUT)))
    cb_pad = jnp.pad(cls_b, (0, 128 - OUT), constant_values=-1e30)
    batch2d = batch.reshape(N // BN_ROWS, 1, BN_ROWS)
    logits = _pool_head_tc(parts[0, :N], parts[1, :N], hws, prev, dinv_col,
                           b[L - 1], batch2d,
                           fc_bn_w, fc_bn_b, fc_bn_m, fc_bn_v,
                           fc_W, fc_b, cw_pad, cb_pad)
    return logits[:, :OUT]
